# Initial kernel scaffold; baseline (speedup 1.0000x reference)
import jax
import jax.numpy as jnp
from jax.experimental import pallas as pl


def kernel(points, gt_boxes, extend_gt_boxes):
    raise NotImplementedError("write your pallas kernel here")


# speedup vs baseline: 14.4994x; 14.4994x over previous
import jax
import jax.numpy as jnp
from jax.experimental import pallas as pl
from jax.experimental.pallas import tpu as pltpu

_LANES = 128
_BLK = 64
_BIG = 1 << 30


def _point_head_kern(bp_ref, keys_ref, x_ref, y_ref, z_ref, out_ref):
    x = x_ref[...]
    y = y_ref[...]
    z = z_ref[...]
    num_boxes = keys_ref.shape[1]

    def body(b, carry):
        keyacc, extacc = carry
        cx = bp_ref[0, b]
        cy = bp_ref[1, b]
        cz = bp_ref[2, b]
        ch = bp_ref[3, b]
        sh = bp_ref[4, b]
        hx = bp_ref[5, b]
        hy = bp_ref[6, b]
        hz = bp_ref[7, b]
        hxe = bp_ref[8, b]
        hye = bp_ref[9, b]
        hze = bp_ref[10, b]
        kb = keys_ref[0, b]
        sx = x - cx
        sy = y - cy
        sz = z - cz
        lx = sx * ch + sy * sh
        ly = sy * ch - sx * sh
        ax = jnp.abs(lx)
        ay = jnp.abs(ly)
        az = jnp.abs(sz)
        in_gt = (ax < hx) & (ay < hy) & (az < hz)
        in_ext = (ax < hxe) & (ay < hye) & (az < hze)
        keyacc = jnp.minimum(keyacc, jnp.where(in_gt, kb, jnp.int32(_BIG)))
        extacc = jnp.where(in_ext, jnp.int32(1), extacc)
        return keyacc, extacc

    keyacc = jnp.full(x.shape, _BIG, jnp.int32)
    extacc = jnp.zeros(x.shape, jnp.int32)
    keyacc, extacc = jax.lax.fori_loop(0, num_boxes, body, (keyacc, extacc))
    fg = keyacc < _BIG
    out_ref[...] = jnp.where(fg, keyacc & 3, -extacc)


def kernel(points, gt_boxes, extend_gt_boxes):
    n = points.shape[0]
    m = gt_boxes.shape[0]
    rows = n // _LANES
    x = points[:, 0].reshape(rows, _LANES)
    y = points[:, 1].reshape(rows, _LANES)
    z = points[:, 2].reshape(rows, _LANES)
    cos_h = jnp.cos(gt_boxes[:, 6])
    sin_h = jnp.sin(gt_boxes[:, 6])
    bp = jnp.stack([
        gt_boxes[:, 0], gt_boxes[:, 1], gt_boxes[:, 2],
        cos_h, sin_h,
        gt_boxes[:, 3] / 2.0, gt_boxes[:, 4] / 2.0, gt_boxes[:, 5] / 2.0,
        extend_gt_boxes[:, 3] / 2.0, extend_gt_boxes[:, 4] / 2.0,
        extend_gt_boxes[:, 5] / 2.0,
    ])
    keys = (jnp.arange(m, dtype=jnp.int32) * 4
            + gt_boxes[:, -1].astype(jnp.int32)).reshape(1, m)
    out = pl.pallas_call(
        _point_head_kern,
        grid=(rows // _BLK,),
        in_specs=[
            pl.BlockSpec(memory_space=pltpu.SMEM),
            pl.BlockSpec(memory_space=pltpu.SMEM),
            pl.BlockSpec((_BLK, _LANES), lambda i: (i, 0)),
            pl.BlockSpec((_BLK, _LANES), lambda i: (i, 0)),
            pl.BlockSpec((_BLK, _LANES), lambda i: (i, 0)),
        ],
        out_specs=pl.BlockSpec((_BLK, _LANES), lambda i: (i, 0)),
        out_shape=jax.ShapeDtypeStruct((rows, _LANES), jnp.int32),
        compiler_params=pltpu.CompilerParams(
            dimension_semantics=("arbitrary",)),
    )(bp, keys, x, y, z)
    return out.reshape(n)


# speedup vs baseline: 19.7617x; 1.3629x over previous
import jax
import jax.numpy as jnp
from jax.experimental import pallas as pl
from jax.experimental.pallas import tpu as pltpu

_LANES = 128
_BLK = 64
_BIG = 1 << 30


def _point_head_kern(bp_ref, keys_ref, x_ref, y_ref, z_ref, out_ref):
    x = x_ref[...]
    y = y_ref[...]
    z = z_ref[...]
    num_boxes = keys_ref.shape[1]

    keyacc = jnp.full(x.shape, _BIG, jnp.int32)
    extacc = jnp.zeros(x.shape, jnp.bool_)
    for b in range(num_boxes):
        cx = bp_ref[0, b]
        cy = bp_ref[1, b]
        cz = bp_ref[2, b]
        ch = bp_ref[3, b]
        sh = bp_ref[4, b]
        hx = bp_ref[5, b]
        hy = bp_ref[6, b]
        hz = bp_ref[7, b]
        hxe = bp_ref[8, b]
        hye = bp_ref[9, b]
        hze = bp_ref[10, b]
        kb = keys_ref[0, b]
        sx = x - cx
        sy = y - cy
        sz = z - cz
        lx = sx * ch + sy * sh
        ly = sy * ch - sx * sh
        ax = jnp.abs(lx)
        ay = jnp.abs(ly)
        az = jnp.abs(sz)
        in_gt = (ax < hx) & (ay < hy) & (az < hz)
        in_ext = (ax < hxe) & (ay < hye) & (az < hze)
        keyacc = jnp.minimum(keyacc, jnp.where(in_gt, kb, jnp.int32(_BIG)))
        extacc = extacc | in_ext
    fg = keyacc < _BIG
    out_ref[...] = jnp.where(fg, keyacc & 3,
                             jnp.where(extacc, jnp.int32(-1), jnp.int32(0)))


def kernel(points, gt_boxes, extend_gt_boxes):
    n = points.shape[0]
    m = gt_boxes.shape[0]
    rows = n // _LANES
    x = points[:, 0].reshape(rows, _LANES)
    y = points[:, 1].reshape(rows, _LANES)
    z = points[:, 2].reshape(rows, _LANES)
    cos_h = jnp.cos(gt_boxes[:, 6])
    sin_h = jnp.sin(gt_boxes[:, 6])
    bp = jnp.stack([
        gt_boxes[:, 0], gt_boxes[:, 1], gt_boxes[:, 2],
        cos_h, sin_h,
        gt_boxes[:, 3] / 2.0, gt_boxes[:, 4] / 2.0, gt_boxes[:, 5] / 2.0,
        extend_gt_boxes[:, 3] / 2.0, extend_gt_boxes[:, 4] / 2.0,
        extend_gt_boxes[:, 5] / 2.0,
    ])
    keys = (jnp.arange(m, dtype=jnp.int32) * 4
            + gt_boxes[:, -1].astype(jnp.int32)).reshape(1, m)
    out = pl.pallas_call(
        _point_head_kern,
        grid=(rows // _BLK,),
        in_specs=[
            pl.BlockSpec(memory_space=pltpu.SMEM),
            pl.BlockSpec(memory_space=pltpu.SMEM),
            pl.BlockSpec((_BLK, _LANES), lambda i: (i, 0)),
            pl.BlockSpec((_BLK, _LANES), lambda i: (i, 0)),
            pl.BlockSpec((_BLK, _LANES), lambda i: (i, 0)),
        ],
        out_specs=pl.BlockSpec((_BLK, _LANES), lambda i: (i, 0)),
        out_shape=jax.ShapeDtypeStruct((rows, _LANES), jnp.int32),
        compiler_params=pltpu.CompilerParams(
            dimension_semantics=("arbitrary",)),
    )(bp, keys, x, y, z)
    return out.reshape(n)


# speedup vs baseline: 20.7489x; 1.0500x over previous
import jax
import jax.numpy as jnp
from jax.experimental import pallas as pl
from jax.experimental.pallas import tpu as pltpu

_LANES = 128
_BLK = 64
_BIG = 1 << 30


def _point_head_kern(bp_ref, keys_ref, pts_ref, out_ref):
    x = pts_ref[0]
    y = pts_ref[1]
    z = pts_ref[2]
    num_boxes = keys_ref.shape[1]

    keyacc = jnp.full(x.shape, _BIG, jnp.int32)
    extacc = jnp.zeros(x.shape, jnp.bool_)
    for b in range(num_boxes):
        cx = bp_ref[0, b]
        cy = bp_ref[1, b]
        cz = bp_ref[2, b]
        ch = bp_ref[3, b]
        sh = bp_ref[4, b]
        hx = bp_ref[5, b]
        hy = bp_ref[6, b]
        hz = bp_ref[7, b]
        hxe = bp_ref[8, b]
        hye = bp_ref[9, b]
        hze = bp_ref[10, b]
        kb = keys_ref[0, b]
        sx = x - cx
        sy = y - cy
        sz = z - cz
        lx = sx * ch + sy * sh
        ly = sy * ch - sx * sh
        ax = jnp.abs(lx)
        ay = jnp.abs(ly)
        az = jnp.abs(sz)
        in_gt = (ax < hx) & (ay < hy) & (az < hz)
        in_ext = (ax < hxe) & (ay < hye) & (az < hze)
        keyacc = jnp.minimum(keyacc, jnp.where(in_gt, kb, jnp.int32(_BIG)))
        extacc = extacc | in_ext
    fg = keyacc < _BIG
    out_ref[...] = jnp.where(fg, keyacc & 3,
                             jnp.where(extacc, jnp.int32(-1), jnp.int32(0)))


def kernel(points, gt_boxes, extend_gt_boxes):
    n = points.shape[0]
    m = gt_boxes.shape[0]
    rows = n // _LANES
    pts = points.T.reshape(3, rows, _LANES)
    cos_h = jnp.cos(gt_boxes[:, 6])
    sin_h = jnp.sin(gt_boxes[:, 6])
    bp = jnp.concatenate([
        gt_boxes[:, 0:3].T,
        cos_h[None], sin_h[None],
        gt_boxes[:, 3:6].T / 2.0,
        extend_gt_boxes[:, 3:6].T / 2.0,
    ], axis=0)
    keys = (jnp.arange(m, dtype=jnp.int32) * 4
            + gt_boxes[:, -1].astype(jnp.int32)).reshape(1, m)
    out = pl.pallas_call(
        _point_head_kern,
        grid=(rows // _BLK,),
        in_specs=[
            pl.BlockSpec(memory_space=pltpu.SMEM),
            pl.BlockSpec(memory_space=pltpu.SMEM),
            pl.BlockSpec((3, _BLK, _LANES), lambda i: (0, i, 0)),
        ],
        out_specs=pl.BlockSpec((_BLK, _LANES), lambda i: (i, 0)),
        out_shape=jax.ShapeDtypeStruct((rows, _LANES), jnp.int32),
        compiler_params=pltpu.CompilerParams(
            dimension_semantics=("arbitrary",)),
    )(bp, keys, pts)
    return out.reshape(n)
